# Initial kernel scaffold; baseline (speedup 1.0000x reference)
import jax
import jax.numpy as jnp
from jax.experimental import pallas as pl


def kernel(x_in, edge_index, edge_attr, params):
    raise NotImplementedError("write your pallas kernel here")


# speedup vs baseline: 2.4717x; 2.4717x over previous
import functools

import jax
import jax.numpy as jnp
from jax import lax
from jax.experimental import pallas as pl
from jax.experimental.pallas import tpu as pltpu
from jax.experimental.pallas import tpu_sc as plsc

N = 10000
NP = 10240
E = 320000
NI = 128
H = 256
C = 7

NC = 2
NS = 16
NW = NC * NS

BN = 512
BE = 512

F32 = jnp.float32


def _mesh():
    return plsc.VectorSubcoreMesh(
        core_axis_name="c", subcore_axis_name="s", num_cores=NC, num_subcores=NS)


def _full(shape):
    return pl.BlockSpec(shape, lambda i: (0,) * len(shape))


def _tc_node_encode(xp, dummy, W1, b1, W2, b2, PA, PS):
    def body(x_ref, dm_ref, w1_ref, b1_ref, w2_ref, b2_ref, pa_ref, ps_ref,
             x_out, ai_out, ts_out):
        xb = x_ref[...]
        inv = xb[:, 0:1] == -999.0
        xc = jnp.where(inv, dm_ref[...], xb)
        h = jnp.maximum(
            jnp.dot(xc, w1_ref[...], preferred_element_type=F32) + b1_ref[...], 0.0)
        xv = jnp.dot(h, w2_ref[...], preferred_element_type=F32) + b2_ref[...]
        x_out[...] = xv
        ai_out[...] = jnp.dot(xv, pa_ref[...], preferred_element_type=F32)
        ts_out[...] = jnp.dot(xv, ps_ref[...], preferred_element_type=F32)

    return pl.pallas_call(
        body,
        grid=(NP // BN,),
        in_specs=[
            pl.BlockSpec((BN, NI), lambda i: (i, 0)),
            _full((1, NI)), _full((NI, H)), _full((1, H)),
            _full((H, H)), _full((1, H)), _full((H, H)), _full((H, 2 * H)),
        ],
        out_specs=[
            pl.BlockSpec((BN, H), lambda i: (i, 0)),
            pl.BlockSpec((BN, H), lambda i: (i, 0)),
            pl.BlockSpec((BN, 2 * H), lambda i: (i, 0)),
        ],
        out_shape=[
            jax.ShapeDtypeStruct((NP, H), F32),
            jax.ShapeDtypeStruct((NP, H), F32),
            jax.ShapeDtypeStruct((NP, 2 * H), F32),
        ],
    )(xp, dummy, W1, b1, W2, b2, PA, PS)


def _tc_attn(ea, gI, gS, Wea, bea, Wa, ca, aW2, ab2):
    def body(ea_ref, gi_ref, gj_ref, wea_ref, bea_ref, wa_ref, ca_ref,
             aw2_ref, ab2_ref, s_out):
        t = jnp.maximum(
            jnp.dot(ea_ref[...], wea_ref[...], preferred_element_type=F32)
            + bea_ref[...], 0.0)
        pre = gi_ref[...] + gj_ref[...] + ca_ref[...] + jnp.dot(
            t, wa_ref[...], preferred_element_type=F32)
        pre = jnp.maximum(pre, 0.0)
        s_out[...] = jnp.dot(pre, aw2_ref[...],
                             preferred_element_type=F32) + ab2_ref[...]

    return pl.pallas_call(
        body,
        grid=(E // BE,),
        in_specs=[
            pl.BlockSpec((BE, 16), lambda i: (i, 0)),
            pl.BlockSpec((BE, H), lambda i: (i, 0)),
            pl.BlockSpec((BE, H), lambda i: (i, 0)),
            _full((16, H)), _full((1, H)), _full((H, H)), _full((1, H)),
            _full((H, 1)), _full((1, 1)),
        ],
        out_specs=pl.BlockSpec((BE, 1), lambda i: (i, 0)),
        out_shape=jax.ShapeDtypeStruct((E, 1), F32),
    )(ea, gI, gS, Wea, bea, Wa, ca, aW2, ab2)


def _tc_msg(ea, gS, alpha, Wea, bea, Wm, cm):
    def body(ea_ref, gx_ref, al_ref, wea_ref, bea_ref, wm_ref, cm_ref,
             w0_out, w1_out):
        t = jnp.maximum(
            jnp.dot(ea_ref[...], wea_ref[...], preferred_element_type=F32)
            + bea_ref[...], 0.0)
        m = jnp.maximum(
            gx_ref[...] + cm_ref[...]
            + jnp.dot(t, wm_ref[...], preferred_element_type=F32), 0.0)
        w = al_ref[...] * m
        w0_out[...] = w[:, :128]
        w1_out[...] = w[:, 128:]

    return pl.pallas_call(
        body,
        grid=(E // BE,),
        in_specs=[
            pl.BlockSpec((BE, 16), lambda i: (i, 0)),
            pl.BlockSpec((BE, H), lambda i: (i, 1)),
            pl.BlockSpec((BE, 1), lambda i: (i, 0)),
            _full((16, H)), _full((1, H)), _full((H, H)), _full((1, H)),
        ],
        out_specs=[
            pl.BlockSpec((BE, 128), lambda i: (i, 0)),
            pl.BlockSpec((BE, 128), lambda i: (i, 0)),
        ],
        out_shape=[
            jax.ShapeDtypeStruct((E, 128), F32),
            jax.ShapeDtypeStruct((E, 128), F32),
        ],
    )(ea, gS, alpha, Wea, bea, Wm, cm)


def _tc_update(S0, S1, den, xprev, mW2a, mW2b, mb2, g, b, head, *proj):
    if not head:
        PA, PS = proj

        def body(s0_ref, s1_ref, den_ref, xp_ref, wa_ref, wb_ref, mb2_ref,
                 g_ref, b_ref, pa_ref, ps_ref, x_out, ai_out, ts_out):
            xn = _update_common(s0_ref, s1_ref, den_ref, xp_ref, wa_ref,
                                wb_ref, mb2_ref, g_ref, b_ref)
            x_out[...] = xn
            ai_out[...] = jnp.dot(xn, pa_ref[...], preferred_element_type=F32)
            ts_out[...] = jnp.dot(xn, ps_ref[...], preferred_element_type=F32)

        extra_in = [_full((H, H)), _full((H, 2 * H))]
        out_specs = [
            pl.BlockSpec((BN, H), lambda i: (i, 0)),
            pl.BlockSpec((BN, H), lambda i: (i, 0)),
            pl.BlockSpec((BN, 2 * H), lambda i: (i, 0)),
        ]
        out_shape = [
            jax.ShapeDtypeStruct((NP, H), F32),
            jax.ShapeDtypeStruct((NP, H), F32),
            jax.ShapeDtypeStruct((NP, 2 * H), F32),
        ]
    else:
        Es, Ed, nhW1, nhb1, nhW2p, nhb2p = proj

        def body(s0_ref, s1_ref, den_ref, xp_ref, wa_ref, wb_ref, mb2_ref,
                 g_ref, b_ref, es_ref, ed_ref, n1_ref, nb1_ref, n2_ref,
                 nb2_ref, ps_out, pd_out, nl_out, nprob_out):
            xn = _update_common(s0_ref, s1_ref, den_ref, xp_ref, wa_ref,
                                wb_ref, mb2_ref, g_ref, b_ref)
            ps_out[...] = jnp.dot(xn, es_ref[...], preferred_element_type=F32)
            pd_out[...] = jnp.dot(xn, ed_ref[...], preferred_element_type=F32)
            hh = jnp.maximum(
                jnp.dot(xn, n1_ref[...], preferred_element_type=F32)
                + nb1_ref[...], 0.0)
            lg = jnp.dot(hh, n2_ref[...], preferred_element_type=F32) + nb2_ref[...]
            nl_out[...] = lg
            mx = jnp.max(lg, axis=1, keepdims=True)
            ex = jnp.exp(lg - mx)
            nprob_out[...] = ex / jnp.sum(ex, axis=1, keepdims=True)

        extra_in = [_full((H, H)), _full((H, H)), _full((H, H)),
                    _full((1, H)), _full((H, 8)), _full((1, 8))]
        out_specs = [
            pl.BlockSpec((BN, H), lambda i: (i, 0)),
            pl.BlockSpec((BN, H), lambda i: (i, 0)),
            pl.BlockSpec((BN, 8), lambda i: (i, 0)),
            pl.BlockSpec((BN, 8), lambda i: (i, 0)),
        ]
        out_shape = [
            jax.ShapeDtypeStruct((NP, H), F32),
            jax.ShapeDtypeStruct((NP, H), F32),
            jax.ShapeDtypeStruct((NP, 8), F32),
            jax.ShapeDtypeStruct((NP, 8), F32),
        ]

    return pl.pallas_call(
        body,
        grid=(NP // BN,),
        in_specs=[
            pl.BlockSpec((BN, 128), lambda i: (i, 0)),
            pl.BlockSpec((BN, 128), lambda i: (i, 0)),
            pl.BlockSpec((1, BN), lambda i: (0, i)),
            pl.BlockSpec((BN, H), lambda i: (i, 0)),
            _full((128, H)), _full((128, H)), _full((1, H)),
            _full((1, H)), _full((1, H)),
        ] + extra_in,
        out_specs=out_specs,
        out_shape=out_shape,
    )(S0, S1, den, xprev, mW2a, mW2b, mb2, g, b, *proj)


def _update_common(s0_ref, s1_ref, den_ref, xp_ref, wa_ref, wb_ref, mb2_ref,
                   g_ref, b_ref):
    u = (jnp.dot(s0_ref[...], wa_ref[...], preferred_element_type=F32)
         + jnp.dot(s1_ref[...], wb_ref[...], preferred_element_type=F32))
    dv = den_ref[0, :]
    sal = dv / (dv + 1e-16)
    v = u + sal[:, None] * mb2_ref[...] + xp_ref[...]
    m = jnp.mean(v, axis=1, keepdims=True)
    var = jnp.mean((v - m) ** 2, axis=1, keepdims=True)
    xn = (v - m) * jax.lax.rsqrt(var + 1e-5) * g_ref[...] + b_ref[...]
    return jnp.maximum(xn, 0.0)


def _tc_edge_head(ea, gPs, gPd, Wea, bea, We, ce, ehW2, ehb2):
    def body(ea_ref, gs_ref, gd_ref, wea_ref, bea_ref, we_ref, ce_ref,
             w2_ref, b2_ref, el_out, ep_out):
        t = jnp.maximum(
            jnp.dot(ea_ref[...], wea_ref[...], preferred_element_type=F32)
            + bea_ref[...], 0.0)
        pre = jnp.maximum(
            gs_ref[...] + gd_ref[...] + ce_ref[...]
            + jnp.dot(t, we_ref[...], preferred_element_type=F32), 0.0)
        z = jnp.dot(pre, w2_ref[...], preferred_element_type=F32) + b2_ref[...]
        el_out[...] = z
        ep_out[...] = jax.nn.sigmoid(z)

    return pl.pallas_call(
        body,
        grid=(E // BE,),
        in_specs=[
            pl.BlockSpec((BE, 16), lambda i: (i, 0)),
            pl.BlockSpec((BE, H), lambda i: (i, 0)),
            pl.BlockSpec((BE, H), lambda i: (i, 0)),
            _full((16, H)), _full((1, H)), _full((H, H)), _full((1, H)),
            _full((H, 1)), _full((1, 1)),
        ],
        out_specs=[
            pl.BlockSpec((BE, 1), lambda i: (i, 0)),
            pl.BlockSpec((BE, 1), lambda i: (i, 0)),
        ],
        out_shape=[
            jax.ShapeDtypeStruct((E, 1), F32),
            jax.ShapeDtypeStruct((E, 1), F32),
        ],
    )(ea, gPs, gPd, Wea, bea, We, ce, ehW2, ehb2)


EPW = E // NW
CH_A = 200
CH_S = 80


def _sc_gather_conv(Ta, Ts, dst, src):
    @functools.partial(
        pl.kernel,
        out_type=[jax.ShapeDtypeStruct((E, H), F32),
                  jax.ShapeDtypeStruct((E, 2 * H), F32)],
        mesh=_mesh(),
        scratch_types=[
            pltpu.VMEM((CH_A,), jnp.int32),
            pltpu.VMEM((CH_S,), jnp.int32),
            pltpu.VMEM((CH_A, H), F32),
            pltpu.VMEM((CH_S, 2 * H), F32),
            pltpu.SemaphoreType.DMA,
        ],
    )
    def k(ta_h, ts_h, dst_h, src_h, gi_h, gs_h, idxa, idxs, bufa, bufs, sem):
        wid = lax.axis_index("s") * NC + lax.axis_index("c")
        base = wid * EPW

        def loop_a(i, _):
            off = base + i * CH_A
            pltpu.sync_copy(dst_h.at[pl.ds(off, CH_A)], idxa)
            pltpu.async_copy(ta_h.at[idxa], bufa, sem).wait()
            pltpu.sync_copy(bufa, gi_h.at[pl.ds(off, CH_A)])
            return 0

        lax.fori_loop(0, EPW // CH_A, loop_a, 0)

        def loop_s(i, _):
            off = base + i * CH_S
            pltpu.sync_copy(src_h.at[pl.ds(off, CH_S)], idxs)
            pltpu.async_copy(ts_h.at[idxs], bufs, sem).wait()
            pltpu.sync_copy(bufs, gs_h.at[pl.ds(off, CH_S)])
            return 0

        lax.fori_loop(0, EPW // CH_S, loop_s, 0)

    return k(Ta, Ts, dst, src)


def _sc_gather_head(Ps, Pd, src, dst):
    @functools.partial(
        pl.kernel,
        out_type=[jax.ShapeDtypeStruct((E, H), F32),
                  jax.ShapeDtypeStruct((E, H), F32)],
        mesh=_mesh(),
        scratch_types=[
            pltpu.VMEM((CH_A,), jnp.int32),
            pltpu.VMEM((CH_A, H), F32),
            pltpu.SemaphoreType.DMA,
        ],
    )
    def k(ps_h, pd_h, src_h, dst_h, gs_h, gd_h, idxa, bufa, sem):
        wid = lax.axis_index("s") * NC + lax.axis_index("c")
        base = wid * EPW

        def loop1(i, _):
            off = base + i * CH_A
            pltpu.sync_copy(src_h.at[pl.ds(off, CH_A)], idxa)
            pltpu.async_copy(ps_h.at[idxa], bufa, sem).wait()
            pltpu.sync_copy(bufa, gs_h.at[pl.ds(off, CH_A)])
            return 0

        lax.fori_loop(0, EPW // CH_A, loop1, 0)

        def loop2(i, _):
            off = base + i * CH_A
            pltpu.sync_copy(dst_h.at[pl.ds(off, CH_A)], idxa)
            pltpu.async_copy(pd_h.at[idxa], bufa, sem).wait()
            pltpu.sync_copy(bufa, gd_h.at[pl.ds(off, CH_A)])
            return 0

        lax.fori_loop(0, EPW // CH_A, loop2, 0)

    return k(Ps, Pd, src, dst)


EPT = E // NS
NPT = NP // NS


def _sc_softmax(s, dst):
    @functools.partial(
        pl.kernel,
        out_type=[jax.ShapeDtypeStruct((E,), F32),
                  jax.ShapeDtypeStruct((NP,), F32)],
        mesh=_mesh(),
        scratch_types=[
            pltpu.VMEM((EPT,), jnp.int32),
            pltpu.VMEM((EPT,), F32),
            pltpu.VMEM((EPT,), F32),
            pltpu.VMEM((NP,), F32),
            pltpu.VMEM((NP,), F32),
            pltpu.VMEM((NPT,), F32),
            pltpu.VMEM_SHARED((NS, NP), F32),
            pltpu.VMEM_SHARED((NP,), F32),
        ],
        compiler_params=pltpu.CompilerParams(needs_layout_passes=False),
    )
    def k(s_h, dst_h, alpha_h, den_h, dst_v, sv, exv, acc, denv, tmp,
          part, dsh):
        cid = lax.axis_index("c")
        sid = lax.axis_index("s")
        base = sid * EPT
        nb = sid * NPT

        @pl.when(cid == 0)
        def _phase_a():
            pltpu.sync_copy(dst_h.at[pl.ds(base, EPT)], dst_v)
            pltpu.sync_copy(s_h.at[pl.ds(base, EPT)], sv)

            def zloop(i, _):
                acc[pl.ds(i * 16, 16)] = jnp.zeros((16,), F32)
                return 0

            lax.fori_loop(0, NP // 16, zloop, 0)

            def eloop(i, _):
                kk = i * 16
                idx = dst_v[pl.ds(kk, 16)]
                ex = jnp.exp(sv[pl.ds(kk, 16)])
                exv[pl.ds(kk, 16)] = ex
                plsc.addupdate_scatter(acc, [idx], ex)
                return 0

            lax.fori_loop(0, EPT // 16, eloop, 0)
            pltpu.sync_copy(acc, part.at[sid])

        plsc.subcore_barrier()

        @pl.when(cid == 0)
        def _phase_b():
            def zloop2(i, _):
                denv[pl.ds(nb + i * 16, 16)] = jnp.zeros((16,), F32)
                return 0

            lax.fori_loop(0, NPT // 16, zloop2, 0)

            def rloop(j, _):
                pltpu.sync_copy(part.at[j, pl.ds(nb, NPT)], tmp)

                def aloop(k2, _):
                    o = nb + k2 * 16
                    denv[pl.ds(o, 16)] = denv[pl.ds(o, 16)] + tmp[pl.ds(k2 * 16, 16)]
                    return 0

                lax.fori_loop(0, NPT // 16, aloop, 0)
                return 0

            lax.fori_loop(0, NS, rloop, 0)
            pltpu.sync_copy(denv.at[pl.ds(nb, NPT)], dsh.at[pl.ds(nb, NPT)])
            pltpu.sync_copy(denv.at[pl.ds(nb, NPT)], den_h.at[pl.ds(nb, NPT)])

        plsc.subcore_barrier()

        @pl.when(cid == 0)
        def _phase_c():
            pltpu.sync_copy(dsh, denv)

            def bloop(i, _):
                kk = i * 16
                idx = dst_v[pl.ds(kk, 16)]
                d = plsc.load_gather(denv, [idx])
                sv[pl.ds(kk, 16)] = exv[pl.ds(kk, 16)] / (d + 1e-16)
                return 0

            lax.fori_loop(0, EPT // 16, bloop, 0)
            pltpu.sync_copy(sv, alpha_h.at[pl.ds(base, EPT)])

    return k(s, dst)


CH_W = 200
ZR = 160


def _sc_scatter(w0, w1, dst, zrows):
    @functools.partial(
        pl.kernel,
        out_type=jax.ShapeDtypeStruct((2 * NP, 128), F32),
        mesh=_mesh(),
        scratch_types=[
            pltpu.VMEM((CH_W,), jnp.int32),
            pltpu.VMEM((CH_W, 128), F32),
            pltpu.VMEM_SHARED((NP, 128), F32),
        ],
    )
    def k(w0_h, w1_h, dst_h, z_h, s_h, idxv, buf, shS):
        cid = lax.axis_index("c")
        sid = lax.axis_index("s")
        rb = sid * NPT

        def zl(i, _):
            pltpu.sync_copy(z_h.at[pl.ds(0, ZR)], buf.at[pl.ds(0, ZR)])
            pltpu.sync_copy(buf.at[pl.ds(0, ZR)], shS.at[pl.ds(rb + i * ZR, ZR)])
            return 0

        lax.fori_loop(0, NPT // ZR, zl, 0)
        plsc.subcore_barrier()

        base = sid * EPT

        def ml0(i, _):
            off = base + i * CH_W
            pltpu.sync_copy(dst_h.at[pl.ds(off, CH_W)], idxv)
            pltpu.sync_copy(w0_h.at[pl.ds(off, CH_W)], buf)
            pltpu.sync_copy(buf, shS.at[idxv], add=True)
            return 0

        def ml1(i, _):
            off = base + i * CH_W
            pltpu.sync_copy(dst_h.at[pl.ds(off, CH_W)], idxv)
            pltpu.sync_copy(w1_h.at[pl.ds(off, CH_W)], buf)
            pltpu.sync_copy(buf, shS.at[idxv], add=True)
            return 0

        @pl.when(cid == 0)
        def _c0():
            lax.fori_loop(0, EPT // CH_W, ml0, 0)

        @pl.when(cid == 1)
        def _c1():
            lax.fori_loop(0, EPT // CH_W, ml1, 0)

        plsc.subcore_barrier()

        def wl(i, _):
            r = rb + i * ZR
            pltpu.sync_copy(shS.at[pl.ds(r, ZR)], buf.at[pl.ds(0, ZR)])
            pltpu.sync_copy(buf.at[pl.ds(0, ZR)], s_h.at[pl.ds(cid * NP + r, ZR)])
            return 0

        lax.fori_loop(0, NPT // ZR, wl, 0)

    return k(w0, w1, dst, zrows)


def kernel(x_in, edge_index, edge_attr, params):
    p = params
    src = edge_index[0]
    dst = edge_index[1]

    eeW2, eeb2 = p['ee_W2'], p['ee_b2']

    def conv_w(pre):
        aW1 = p[pre + '_aW1']
        A_i, A_j, A_e = aW1[:H], aW1[H:2 * H], aW1[2 * H:]
        mW1 = p[pre + '_mW1']
        M_x, M_e = mW1[:H], mW1[H:]
        Wa = eeW2 @ A_e
        ca = (eeb2 @ A_e + p[pre + '_ab1'])[None, :]
        Wm = eeW2 @ M_e
        cm = (eeb2 @ M_e + p[pre + '_mb1'])[None, :]
        PA = A_i
        PS = jnp.concatenate([A_j, M_x], axis=1)
        return Wa, ca, Wm, cm, PA, PS

    Wa1, ca1, Wm1, cm1, PA1, PS1 = conv_w('c1')
    Wa2, ca2, Wm2, cm2, PA2, PS2 = conv_w('c2')
    ehW1 = p['eh_W1']
    E_s, E_d, E_e = ehW1[:H], ehW1[H:2 * H], ehW1[2 * H:]
    We = eeW2 @ E_e
    ce = (eeb2 @ E_e + p['eh_b1'])[None, :]

    mW2_1a, mW2_1b = p['c1_mW2'][:128], p['c1_mW2'][128:]
    mW2_2a, mW2_2b = p['c2_mW2'][:128], p['c2_mW2'][128:]
    nhW2p = jnp.pad(p['nh_W2'], ((0, 0), (0, 1)))
    nhb2p = jnp.pad(p['nh_b2'], (0, 1), constant_values=-1e9)[None, :]

    xp = jnp.pad(x_in, ((0, NP - N), (0, 0)))
    zrows = jnp.zeros((ZR, 128), F32)

    x0, aI1, TS1 = _tc_node_encode(
        xp, p['dummy'][None, :], p['ne_W1'], p['ne_b1'][None, :],
        p['ne_W2'], p['ne_b2'][None, :], PA1, PS1)

    def conv(xprev, aI, TS, Wa, ca, Wm, cm, aW2, ab2, mW2a, mW2b, mb2, g, b,
             head, proj):
        gI, gS = _sc_gather_conv(aI, TS, dst, src)
        s = _tc_attn(edge_attr, gI, gS, p['ee_W1'], p['ee_b1'][None, :],
                     Wa, ca, aW2, ab2[None, :])
        alpha, den = _sc_softmax(s.reshape(E), dst)
        w0, w1 = _tc_msg(edge_attr, gS, alpha.reshape(E, 1),
                         p['ee_W1'], p['ee_b1'][None, :], Wm, cm)
        S = _sc_scatter(w0, w1, dst, zrows)
        return _tc_update(S[:NP], S[NP:], den.reshape(1, NP), xprev,
                          mW2a, mW2b, mb2[None, :], g[None, :], b[None, :],
                          head, *proj)

    x1, aI2, TS2 = conv(x0, aI1, TS1, Wa1, ca1, Wm1, cm1,
                        p['c1_aW2'], p['c1_ab2'], mW2_1a, mW2_1b,
                        p['c1_mb2'], p['ln1_g'], p['ln1_b'],
                        False, (PA2, PS2))

    Ps, Pd, nlp, npp = conv(x1, aI2, TS2, Wa2, ca2, Wm2, cm2,
                            p['c2_aW2'], p['c2_ab2'], mW2_2a, mW2_2b,
                            p['c2_mb2'], p['ln2_g'], p['ln2_b'],
                            True, (E_s, E_d, p['nh_W1'], p['nh_b1'][None, :],
                                   nhW2p, nhb2p))

    gPs, gPd = _sc_gather_head(Ps, Pd, src, dst)
    el, ep = _tc_edge_head(edge_attr, gPs, gPd, p['ee_W1'],
                           p['ee_b1'][None, :], We, ce,
                           p['eh_W2'], p['eh_b2'][None, :])

    node_logits = nlp[:N, :C]
    node_probs = npp[:N, :C]
    return (node_logits, el, node_probs, ep)
